# baseline (speedup 1.0000x reference)
import jax
import jax.numpy as jnp
from jax.experimental import pallas as pl
from jax.experimental.pallas import tpu as pltpu

_TILE_B = 512


def _make_body(tile_b, ncopies):
    def body(tab_ref, out_ref, scratch_ref, sems):
        scratch_ref[...] = jnp.broadcast_to(tab_ref[...], scratch_ref.shape)
        for i in range(ncopies):
            pltpu.make_async_copy(
                scratch_ref,
                out_ref.at[pl.ds(i * tile_b, tile_b), :],
                sems.at[i],
            ).start()
        for i in range(ncopies):
            pltpu.make_async_copy(
                scratch_ref,
                out_ref.at[pl.ds(i * tile_b, tile_b), :],
                sems.at[i],
            ).wait()

    return body


def kernel(x, pos_table):
    batch = x.shape[0]
    seq_len = x.shape[1]
    emb = pos_table.shape[1]
    flat = seq_len * emb
    tab = pos_table[:seq_len].reshape(1, flat)

    tile_b = _TILE_B if batch % _TILE_B == 0 else batch
    ncopies = batch // tile_b

    out = pl.pallas_call(
        _make_body(tile_b, ncopies),
        in_specs=[pl.BlockSpec((1, flat), lambda: (0, 0))],
        out_specs=pl.BlockSpec(memory_space=pltpu.MemorySpace.HBM),
        out_shape=jax.ShapeDtypeStruct((batch, flat), pos_table.dtype),
        scratch_shapes=[
            pltpu.VMEM((tile_b, flat), pos_table.dtype),
            pltpu.SemaphoreType.DMA((ncopies,)),
        ],
    )(tab)
    return out.reshape(batch, seq_len, emb)
